# baseline (speedup 1.0000x reference)
import jax
import jax.numpy as jnp
from jax import lax
from jax.experimental import pallas as pl
from jax.experimental.pallas import tpu as pltpu
from jax.experimental.pallas import tpu_sc as plsc

NC = 2
NS = 16
NW = NC * NS
CH = 128
NBUF = 3
NNODE = 10000
FDIM = 128
RPS = 632
RPS_LAST = NNODE - (NS - 1) * RPS

NCHUNK = -(-320000 // CH)
QBASE = NCHUNK // NW
QREM = NCHUNK % NW


def _seg_loop(with_cnt, q, off_e, feat, edges, acc_sh, cnt_sh, ones_v,
              isrc, idst, rows, isem, dsem, gsem, ssem, csem):
  nb = len(isrc)
  ng = q // nb

  def _isrc(j, b):
    return pltpu.make_async_copy(
        edges.at[0, pl.ds(off_e + j * CH, CH)], isrc[b], isem.at[b])

  def _idst(j, b):
    return pltpu.make_async_copy(
        edges.at[pl.ds(1, 1), pl.ds(off_e + j * CH, CH)], idst[b],
        dsem.at[b])

  half = CH // 2

  def _gather_d(b, o):
    return pltpu.make_async_copy(feat.at[isrc[b].at[pl.ds(o, half)]],
                                 rows[b].at[pl.ds(o, half)], gsem.at[b])

  def _gather_start(b):
    _gather_d(b, 0).start()
    _gather_d(b, half).start()

  def _gather_wait(b):
    _gather_d(b, 0).wait()
    _gather_d(b, half).wait()

  def _scat(b):
    return pltpu.async_copy(rows[b], acc_sh.at[idst[b].at[0]], ssem.at[b],
                            add=True)

  def _cnt(b):
    return pltpu.async_copy(ones_v, cnt_sh.at[idst[b].at[0]], csem.at[b],
                            add=True)

  for b in range(nb):
    _isrc(b, b).start()
    _idst(b, b).start()
    _isrc(b, b).wait()
    _gather_start(b)

  def group(t, carry):
    base = t * nb
    descs = []
    for b in range(nb):
      j = base + b
      _gather_wait(b)
      _idst(j, b).wait()
      sd = _scat(b)
      cd = _cnt(b) if with_cnt else None
      descs.append((sd, cd))
      _isrc((j + nb) % q, b).start()
    for b in range(nb):
      j = base + b
      sd, cd = descs[b]
      sd.wait()
      if cd is not None:
        cd.wait()
      _idst((j + nb) % q, b).start()
      _isrc(j, b).wait()
      _gather_start(b)
    return carry

  lax.fori_loop(0, ng, group, 0)
  for b in range(nb):
    _gather_wait(b)
    _idst(b, b).wait()

  def tail(j, carry):
    _isrc(j, 0).start()
    _idst(j, 0).start()
    _isrc(j, 0).wait()
    _idst(j, 0).wait()
    _gather_start(0)
    _gather_wait(0)
    _scat(0).wait()
    if with_cnt:
      _cnt(0).wait()
    return carry

  lax.fori_loop(ng * nb, q, tail, 0)


def _tile_quota(c, s):
  w = s * NC + c
  q = QBASE + jnp.where(w < QREM, 1, 0)
  off_e = (w * QBASE + jnp.minimum(w, QREM)) * CH
  return q, off_e


def _slab(s, copy):
  base = s * RPS

  @pl.when(s < NS - 1)
  def _():
    copy(pl.ds(base, RPS))

  @pl.when(s == NS - 1)
  def _():
    copy(pl.ds(base, RPS_LAST))


def _seg_sum_cnt_body(feat, edges, z2d, z1d, ones_h, out_p, out_c,
                      acc_sh, cnt_sh, ones_v,
                      isrc0, isrc1, isrc2, idst0, idst1, idst2,
                      rows0, rows1, rows2,
                      isem, dsem, gsem, ssem, csem):
  c = lax.axis_index("c")
  s = lax.axis_index("s")
  q, off_e = _tile_quota(c, s)
  _slab(s, lambda d: pltpu.sync_copy(z2d.at[d], acc_sh.at[d]))

  @pl.when(s == 0)
  def _():
    pltpu.sync_copy(z1d, cnt_sh)

  pltpu.sync_copy(ones_h, ones_v)
  plsc.subcore_barrier()
  _seg_loop(True, q, off_e, feat, edges, acc_sh, cnt_sh, ones_v,
            (isrc0, isrc1, isrc2), (idst0, idst1, idst2),
            (rows0, rows1, rows2), isem, dsem, gsem, ssem, csem)
  plsc.subcore_barrier()
  _slab(s, lambda d: pltpu.sync_copy(acc_sh.at[d], out_p.at[c, d]))

  @pl.when(s == 0)
  def _():
    pltpu.sync_copy(cnt_sh, out_c.at[c])


def _seg_sum_body(feat, edges, z2d, out_p,
                  acc_sh,
                  isrc0, isrc1, isrc2, idst0, idst1, idst2,
                  rows0, rows1, rows2,
                  isem, dsem, gsem, ssem):
  c = lax.axis_index("c")
  s = lax.axis_index("s")
  q, off_e = _tile_quota(c, s)
  _slab(s, lambda d: pltpu.sync_copy(z2d.at[d], acc_sh.at[d]))
  plsc.subcore_barrier()
  _seg_loop(False, q, off_e, feat, edges, acc_sh, None, None,
            (isrc0, isrc1, isrc2), (idst0, idst1, idst2),
            (rows0, rows1, rows2), isem, dsem, gsem, ssem, None)
  plsc.subcore_barrier()
  _slab(s, lambda d: pltpu.sync_copy(acc_sh.at[d], out_p.at[c, d]))


def _ring_bufs(nb):
  return (
      [pltpu.VMEM((CH,), jnp.int32) for _ in range(nb)] +
      [pltpu.VMEM((1, CH), jnp.int32) for _ in range(nb)] +
      [pltpu.VMEM((CH, FDIM), jnp.float32) for _ in range(nb)])


def _make_seg_kernels():
  mesh = plsc.VectorSubcoreMesh(core_axis_name="c", subcore_axis_name="s")
  seg_cnt = pl.kernel(
      _seg_sum_cnt_body,
      out_type=(jax.ShapeDtypeStruct((NC, NNODE, FDIM), jnp.float32),
                jax.ShapeDtypeStruct((NC, NNODE), jnp.float32)),
      mesh=mesh,
      scratch_types=[
          pltpu.VMEM_SHARED((NNODE, FDIM), jnp.float32),
          pltpu.VMEM_SHARED((NNODE,), jnp.float32),
          pltpu.VMEM((CH,), jnp.float32),
      ] + _ring_bufs(NBUF) + [
          pltpu.SemaphoreType.DMA((NBUF,)),
          pltpu.SemaphoreType.DMA((NBUF,)),
          pltpu.SemaphoreType.DMA((NBUF,)),
          pltpu.SemaphoreType.DMA((NBUF,)),
          pltpu.SemaphoreType.DMA((NBUF,)),
      ],
      name="sage_seg_sum_cnt",
  )
  seg = pl.kernel(
      _seg_sum_body,
      out_type=jax.ShapeDtypeStruct((NC, NNODE, FDIM), jnp.float32),
      mesh=mesh,
      scratch_types=[
          pltpu.VMEM_SHARED((NNODE, FDIM), jnp.float32),
      ] + _ring_bufs(NBUF) + [
          pltpu.SemaphoreType.DMA((NBUF,)),
          pltpu.SemaphoreType.DMA((NBUF,)),
          pltpu.SemaphoreType.DMA((NBUF,)),
          pltpu.SemaphoreType.DMA((NBUF,)),
      ],
      name="sage_seg_sum",
  )
  return seg_cnt, seg


BR = 1000


def _pre_body(x_ref, ws_ref, b_ref, o_ref):
  o_ref[...] = (jnp.dot(x_ref[...], ws_ref[...],
                        preferred_element_type=jnp.float32) + b_ref[...])


def _pre(x, ws, b):
  return pl.pallas_call(
      _pre_body,
      grid=(NNODE // BR,),
      in_specs=[
          pl.BlockSpec((BR, FDIM), lambda i: (i, 0)),
          pl.BlockSpec((FDIM, FDIM), lambda i: (0, 0)),
          pl.BlockSpec((1, FDIM), lambda i: (0, 0)),
      ],
      out_specs=pl.BlockSpec((BR, FDIM), lambda i: (i, 0)),
      out_shape=jax.ShapeDtypeStruct((NNODE, FDIM), jnp.float32),
  )(x, ws, b.reshape(1, FDIM))


def _comb_body(pre_ref, p_ref, c_ref, wn_ref, o_ref):
  agg = (p_ref[0] + p_ref[1]) / jnp.maximum(c_ref[0] + c_ref[1], 1.0)
  o_ref[...] = jnp.maximum(
      pre_ref[...] + jnp.dot(agg, wn_ref[...],
                             preferred_element_type=jnp.float32), 0.0)


def _comb(pre, p, cnt3, wn):
  return pl.pallas_call(
      _comb_body,
      grid=(NNODE // BR,),
      in_specs=[
          pl.BlockSpec((BR, FDIM), lambda i: (i, 0)),
          pl.BlockSpec((NC, BR, FDIM), lambda i: (0, i, 0)),
          pl.BlockSpec((NC, BR, 1), lambda i: (0, i, 0)),
          pl.BlockSpec((FDIM, FDIM), lambda i: (0, 0)),
      ],
      out_specs=pl.BlockSpec((BR, FDIM), lambda i: (i, 0)),
      out_shape=jax.ShapeDtypeStruct((NNODE, FDIM), jnp.float32),
  )(pre, p, cnt3, wn)


def kernel(x, edge_index, W_self1, W_neigh1, b1, W_self2, W_neigh2, b2):
  z2d = jnp.zeros((NNODE, FDIM), jnp.float32)
  z1d = jnp.zeros((NNODE,), jnp.float32)
  ones_h = jnp.ones((CH,), jnp.float32)

  seg_cnt, seg = _make_seg_kernels()
  p1, cnts = seg_cnt(x, edge_index, z2d, z1d, ones_h)
  pre1 = _pre(x, W_self1, b1)
  cnt3 = cnts.reshape(NC, NNODE, 1)
  h = _comb(pre1, p1, cnt3, W_neigh1)
  p2 = seg(h, edge_index, z2d)
  pre2 = _pre(h, W_self2, b2)
  return _comb(pre2, p2, cnt3, W_neigh2)


# speedup vs baseline: 1.0159x; 1.0159x over previous
import jax
import jax.numpy as jnp
from jax import lax
from jax.experimental import pallas as pl
from jax.experimental.pallas import tpu as pltpu
from jax.experimental.pallas import tpu_sc as plsc

NC = 2
NS = 16
NW = NC * NS
CH = 128
NBUF = 3
NNODE = 10000
FDIM = 128
RPS = 632
RPS_LAST = NNODE - (NS - 1) * RPS

NCHUNK = -(-320000 // CH)
QBASE = NCHUNK // NW
QREM = NCHUNK % NW


def _seg_loop(with_cnt, q, off_e, feat, edges, acc_sh, cnt_sh, ones_v,
              isrc, idst, rows, isem, dsem, gsem, ssem, csem):
  nb = len(isrc)
  ng = q // nb

  def _isrc(j, b):
    return pltpu.make_async_copy(
        edges.at[0, pl.ds(off_e + j * CH, CH)], isrc[b], isem.at[b])

  def _idst(j, b):
    return pltpu.make_async_copy(
        edges.at[pl.ds(1, 1), pl.ds(off_e + j * CH, CH)], idst[b],
        dsem.at[b])

  def _gather(b):
    return pltpu.make_async_copy(feat.at[isrc[b]], rows[b], gsem.at[b])

  def _gather_start(b):
    _gather(b).start()

  def _gather_wait(b):
    _gather(b).wait()

  def _scat(b):
    return pltpu.async_copy(rows[b], acc_sh.at[idst[b].at[0]], ssem.at[b],
                            add=True)

  def _cnt(b):
    return pltpu.async_copy(ones_v, cnt_sh.at[idst[b].at[0]], csem.at[b],
                            add=True)

  for b in range(nb):
    _isrc(b, b).start()
    _idst(b, b).start()
    _isrc(b, b).wait()
    _gather_start(b)

  def group(t, carry):
    base = t * nb
    descs = []
    for b in range(nb):
      j = base + b
      _gather_wait(b)
      _idst(j, b).wait()
      sd = _scat(b)
      cd = _cnt(b) if with_cnt else None
      descs.append((sd, cd))
      _isrc((j + nb) % q, b).start()
    for b in range(nb):
      j = base + b
      sd, cd = descs[b]
      sd.wait()
      if cd is not None:
        cd.wait()
      _idst((j + nb) % q, b).start()
      _isrc(j, b).wait()
      _gather_start(b)
    return carry

  lax.fori_loop(0, ng, group, 0)
  for b in range(nb):
    _gather_wait(b)
    _idst(b, b).wait()

  def tail(j, carry):
    _isrc(j, 0).start()
    _idst(j, 0).start()
    _isrc(j, 0).wait()
    _idst(j, 0).wait()
    _gather_start(0)
    _gather_wait(0)
    _scat(0).wait()
    if with_cnt:
      _cnt(0).wait()
    return carry

  lax.fori_loop(ng * nb, q, tail, 0)


def _tile_quota(c, s):
  w = s * NC + c
  q = QBASE + jnp.where(w < QREM, 1, 0)
  off_e = (w * QBASE + jnp.minimum(w, QREM)) * CH
  return q, off_e


def _slab(s, copy):
  base = s * RPS

  @pl.when(s < NS - 1)
  def _():
    copy(pl.ds(base, RPS))

  @pl.when(s == NS - 1)
  def _():
    copy(pl.ds(base, RPS_LAST))


def _seg_sum_cnt_body(feat, edges, z2d, z1d, ones_h, out_p, out_c,
                      acc_sh, cnt_sh, ones_v,
                      isrc0, isrc1, isrc2, idst0, idst1, idst2,
                      rows0, rows1, rows2,
                      isem, dsem, gsem, ssem, csem):
  c = lax.axis_index("c")
  s = lax.axis_index("s")
  q, off_e = _tile_quota(c, s)
  _slab(s, lambda d: pltpu.sync_copy(z2d.at[d], acc_sh.at[d]))

  @pl.when(s == 0)
  def _():
    pltpu.sync_copy(z1d, cnt_sh)

  pltpu.sync_copy(ones_h, ones_v)
  plsc.subcore_barrier()
  _seg_loop(True, q, off_e, feat, edges, acc_sh, cnt_sh, ones_v,
            (isrc0, isrc1, isrc2), (idst0, idst1, idst2),
            (rows0, rows1, rows2), isem, dsem, gsem, ssem, csem)
  plsc.subcore_barrier()
  _slab(s, lambda d: pltpu.sync_copy(acc_sh.at[d], out_p.at[c, d]))

  @pl.when(s == 0)
  def _():
    pltpu.sync_copy(cnt_sh, out_c.at[c])


def _seg_sum_body(feat, edges, z2d, out_p,
                  acc_sh,
                  isrc0, isrc1, isrc2, idst0, idst1, idst2,
                  rows0, rows1, rows2,
                  isem, dsem, gsem, ssem):
  c = lax.axis_index("c")
  s = lax.axis_index("s")
  q, off_e = _tile_quota(c, s)
  _slab(s, lambda d: pltpu.sync_copy(z2d.at[d], acc_sh.at[d]))
  plsc.subcore_barrier()
  _seg_loop(False, q, off_e, feat, edges, acc_sh, None, None,
            (isrc0, isrc1, isrc2), (idst0, idst1, idst2),
            (rows0, rows1, rows2), isem, dsem, gsem, ssem, None)
  plsc.subcore_barrier()
  _slab(s, lambda d: pltpu.sync_copy(acc_sh.at[d], out_p.at[c, d]))


def _ring_bufs(nb):
  return (
      [pltpu.VMEM((CH,), jnp.int32) for _ in range(nb)] +
      [pltpu.VMEM((1, CH), jnp.int32) for _ in range(nb)] +
      [pltpu.VMEM((CH, FDIM), jnp.float32) for _ in range(nb)])


def _make_seg_kernels():
  mesh = plsc.VectorSubcoreMesh(core_axis_name="c", subcore_axis_name="s")
  seg_cnt = pl.kernel(
      _seg_sum_cnt_body,
      out_type=(jax.ShapeDtypeStruct((NC, NNODE, FDIM), jnp.float32),
                jax.ShapeDtypeStruct((NC, NNODE), jnp.float32)),
      mesh=mesh,
      scratch_types=[
          pltpu.VMEM_SHARED((NNODE, FDIM), jnp.float32),
          pltpu.VMEM_SHARED((NNODE,), jnp.float32),
          pltpu.VMEM((CH,), jnp.float32),
      ] + _ring_bufs(NBUF) + [
          pltpu.SemaphoreType.DMA((NBUF,)),
          pltpu.SemaphoreType.DMA((NBUF,)),
          pltpu.SemaphoreType.DMA((NBUF,)),
          pltpu.SemaphoreType.DMA((NBUF,)),
          pltpu.SemaphoreType.DMA((NBUF,)),
      ],
      name="sage_seg_sum_cnt",
  )
  seg = pl.kernel(
      _seg_sum_body,
      out_type=jax.ShapeDtypeStruct((NC, NNODE, FDIM), jnp.float32),
      mesh=mesh,
      scratch_types=[
          pltpu.VMEM_SHARED((NNODE, FDIM), jnp.float32),
      ] + _ring_bufs(NBUF) + [
          pltpu.SemaphoreType.DMA((NBUF,)),
          pltpu.SemaphoreType.DMA((NBUF,)),
          pltpu.SemaphoreType.DMA((NBUF,)),
          pltpu.SemaphoreType.DMA((NBUF,)),
      ],
      name="sage_seg_sum",
  )
  return seg_cnt, seg


BR = 2000


def _pre_body(x_ref, ws_ref, b_ref, o_ref):
  o_ref[...] = (jnp.dot(x_ref[...], ws_ref[...],
                        preferred_element_type=jnp.float32) + b_ref[...])


def _pre(x, ws, b):
  return pl.pallas_call(
      _pre_body,
      grid=(NNODE // BR,),
      in_specs=[
          pl.BlockSpec((BR, FDIM), lambda i: (i, 0)),
          pl.BlockSpec((FDIM, FDIM), lambda i: (0, 0)),
          pl.BlockSpec((1, FDIM), lambda i: (0, 0)),
      ],
      out_specs=pl.BlockSpec((BR, FDIM), lambda i: (i, 0)),
      out_shape=jax.ShapeDtypeStruct((NNODE, FDIM), jnp.float32),
  )(x, ws, b.reshape(1, FDIM))


def _comb_body(pre_ref, p_ref, c_ref, wn_ref, o_ref):
  agg = (p_ref[0] + p_ref[1]) / jnp.maximum(c_ref[0] + c_ref[1], 1.0)
  o_ref[...] = jnp.maximum(
      pre_ref[...] + jnp.dot(agg, wn_ref[...],
                             preferred_element_type=jnp.float32), 0.0)


def _comb(pre, p, cnt3, wn):
  return pl.pallas_call(
      _comb_body,
      grid=(NNODE // BR,),
      in_specs=[
          pl.BlockSpec((BR, FDIM), lambda i: (i, 0)),
          pl.BlockSpec((NC, BR, FDIM), lambda i: (0, i, 0)),
          pl.BlockSpec((NC, BR, 1), lambda i: (0, i, 0)),
          pl.BlockSpec((FDIM, FDIM), lambda i: (0, 0)),
      ],
      out_specs=pl.BlockSpec((BR, FDIM), lambda i: (i, 0)),
      out_shape=jax.ShapeDtypeStruct((NNODE, FDIM), jnp.float32),
  )(pre, p, cnt3, wn)


def kernel(x, edge_index, W_self1, W_neigh1, b1, W_self2, W_neigh2, b2):
  z2d = jnp.zeros((NNODE, FDIM), jnp.float32)
  z1d = jnp.zeros((NNODE,), jnp.float32)
  ones_h = jnp.ones((CH,), jnp.float32)

  seg_cnt, seg = _make_seg_kernels()
  p1, cnts = seg_cnt(x, edge_index, z2d, z1d, ones_h)
  pre1 = _pre(x, W_self1, b1)
  cnt3 = cnts.reshape(NC, NNODE, 1)
  h = _comb(pre1, p1, cnt3, W_neigh1)
  p2 = seg(h, edge_index, z2d)
  pre2 = _pre(h, W_self2, b2)
  return _comb(pre2, p2, cnt3, W_neigh2)
